# initial kernel scaffold (unmeasured)
import jax
import jax.numpy as jnp
from jax import lax
from jax.experimental import pallas as pl
from jax.experimental.pallas import tpu as pltpu

N_DEV = 8


def _gelu(y):
    c = 0.7978845608028654
    return 0.5 * y * (1.0 + jnp.tanh(c * (y + 0.044715 * y * y * y)))


def kernel(x, w_mat):
    m_global, k_shard = x.shape
    _, n = w_mat.shape
    m_out = m_global // N_DEV

    def body(x_hbm, w_ref, out_hbm, x_vmem, send_ref, recv_ref,
             load_sem, store_sem, send_sems, recv_sems, credit_sem):
        my = lax.axis_index("i")
        left = jax.lax.rem(my + N_DEV - 1, N_DEV)
        right = jax.lax.rem(my + 1, N_DEV)

        barrier = pltpu.get_barrier_semaphore()
        for nbr in (left, right):
            pl.semaphore_signal(
                barrier, inc=1, device_id=(nbr,),
                device_id_type=pl.DeviceIdType.MESH,
            )
        pl.semaphore_wait(barrier, 2)

        def load_chunk(c):
            cp = pltpu.make_async_copy(
                x_hbm.at[pl.ds(c * m_out, m_out), :], x_vmem, load_sem
            )
            cp.start()
            return cp

        load_chunk(jax.lax.rem(my + N_DEV - 1, N_DEV)).wait()
        send_ref[...] = jnp.dot(
            x_vmem[...], w_ref[...], preferred_element_type=jnp.float32
        )

        for s in range(N_DEV - 1):
            if s >= 1:
                pl.semaphore_wait(credit_sem, 1)
            rdma = pltpu.make_async_remote_copy(
                src_ref=send_ref,
                dst_ref=recv_ref,
                send_sem=send_sems.at[s],
                recv_sem=recv_sems.at[s],
                device_id=(right,),
                device_id_type=pl.DeviceIdType.MESH,
            )
            rdma.start()
            c_next = jax.lax.rem(my + 2 * N_DEV - 2 - s, N_DEV)
            load_chunk(c_next).wait()
            rdma.wait()
            partial = jnp.dot(
                x_vmem[...], w_ref[...], preferred_element_type=jnp.float32
            )
            if s < N_DEV - 2:
                send_ref[...] = recv_ref[...] + partial
                pl.semaphore_signal(
                    credit_sem, inc=1, device_id=(left,),
                    device_id_type=pl.DeviceIdType.MESH,
                )
            else:
                send_ref[...] = _gelu(recv_ref[...] + partial)
                out_cp = pltpu.make_async_copy(send_ref, out_hbm, store_sem)
                out_cp.start()
                out_cp.wait()

    return pl.pallas_call(
        body,
        out_shape=jax.ShapeDtypeStruct((m_out, n), jnp.float32),
        in_specs=[
            pl.BlockSpec(memory_space=pltpu.ANY),
            pl.BlockSpec(memory_space=pltpu.VMEM),
        ],
        out_specs=pl.BlockSpec(memory_space=pltpu.ANY),
        scratch_shapes=[
            pltpu.VMEM((m_out, k_shard), jnp.float32),
            pltpu.VMEM((m_out, n), jnp.float32),
            pltpu.VMEM((m_out, n), jnp.float32),
            pltpu.SemaphoreType.DMA,
            pltpu.SemaphoreType.DMA,
            pltpu.SemaphoreType.DMA((N_DEV - 1,)),
            pltpu.SemaphoreType.DMA((N_DEV - 1,)),
            pltpu.SemaphoreType.REGULAR,
        ],
        compiler_params=pltpu.CompilerParams(collective_id=0),
    )(x, w_mat)


# baseline (device time: 1418242 ns/iter reference)
import jax
import jax.numpy as jnp
from jax import lax
from jax.experimental import pallas as pl
from jax.experimental.pallas import tpu as pltpu

N_DEV = 8
N_HALF = 2
N_STRIP = 1024


def _gelu(y):
    c = 0.7978845608028654
    return 0.5 * y * (1.0 + jnp.tanh(c * (y + 0.044715 * y * y * y)))


def kernel(x, w_mat):
    m_global, k_shard = x.shape
    _, n = w_mat.shape
    m_out = m_global // N_DEV
    n_half = n // N_HALF

    def body(x_hbm, w_hbm, out_ref, x_vmem, w_vmem, send_ref, recv_ref,
             load_sem, w_sem, send_sems, recv_sems, credit_sem):
        my = lax.axis_index("i")
        left = lax.rem(my + N_DEV - 1, N_DEV)
        right = lax.rem(my + 1, N_DEV)

        barrier = pltpu.get_barrier_semaphore()
        for nbr in (left, right):
            pl.semaphore_signal(
                barrier, inc=1, device_id=(nbr,),
                device_id_type=pl.DeviceIdType.MESH,
            )
        pl.semaphore_wait(barrier, 2)

        w_cp = pltpu.make_async_copy(w_hbm, w_vmem, w_sem)
        w_cp.start()
        w_cp.wait()

        def load_chunk(c):
            cp = pltpu.make_async_copy(
                x_hbm.at[pl.ds(c * m_out, m_out), :], x_vmem, load_sem
            )
            cp.start()
            return cp

        for h in range(N_HALF):
            col0 = h * n_half

            def gemm_accum(add_recv, apply_gelu, h=h, col0=col0):
                for t in range(n_half // N_STRIP):
                    p = jnp.dot(
                        x_vmem[...],
                        w_vmem[:, col0 + t * N_STRIP:col0 + (t + 1) * N_STRIP],
                        preferred_element_type=jnp.float32,
                    )
                    loc = pl.ds(t * N_STRIP, N_STRIP)
                    if add_recv:
                        p = recv_ref[:, loc] + p
                    if apply_gelu:
                        out_ref[:, pl.ds(col0 + t * N_STRIP, N_STRIP)] = _gelu(p)
                    else:
                        send_ref[:, loc] = p

            load_chunk(lax.rem(my + N_DEV - 1, N_DEV)).wait()
            gemm_accum(add_recv=False, apply_gelu=False)

            for s in range(N_DEV - 1):
                if s >= 1 or h >= 1:
                    pl.semaphore_wait(credit_sem, 1)
                rdma = pltpu.make_async_remote_copy(
                    src_ref=send_ref,
                    dst_ref=recv_ref,
                    send_sem=send_sems.at[h, s],
                    recv_sem=recv_sems.at[h, s],
                    device_id=(right,),
                    device_id_type=pl.DeviceIdType.MESH,
                )
                rdma.start()
                c_next = lax.rem(my + 2 * N_DEV - 2 - s, N_DEV)
                load_chunk(c_next).wait()
                rdma.wait()
                if s < N_DEV - 2:
                    gemm_accum(add_recv=True, apply_gelu=False)
                    pl.semaphore_signal(
                        credit_sem, inc=1, device_id=(left,),
                        device_id_type=pl.DeviceIdType.MESH,
                    )
                else:
                    gemm_accum(add_recv=True, apply_gelu=True)
                    if h < N_HALF - 1:
                        pl.semaphore_signal(
                            credit_sem, inc=1, device_id=(left,),
                            device_id_type=pl.DeviceIdType.MESH,
                        )

    return pl.pallas_call(
        body,
        out_shape=jax.ShapeDtypeStruct((m_out, n), jnp.float32),
        in_specs=[
            pl.BlockSpec(memory_space=pltpu.MemorySpace.HBM),
            pl.BlockSpec(memory_space=pltpu.MemorySpace.HBM),
        ],
        out_specs=pl.BlockSpec(memory_space=pltpu.VMEM),
        scratch_shapes=[
            pltpu.VMEM((m_out, k_shard), jnp.float32),
            pltpu.VMEM((k_shard, n), jnp.float32),
            pltpu.VMEM((m_out, n_half), jnp.float32),
            pltpu.VMEM((m_out, n_half), jnp.float32),
            pltpu.SemaphoreType.DMA,
            pltpu.SemaphoreType.DMA,
            pltpu.SemaphoreType.DMA((N_HALF, N_DEV - 1)),
            pltpu.SemaphoreType.DMA((N_HALF, N_DEV - 1)),
            pltpu.SemaphoreType.REGULAR,
        ],
        compiler_params=pltpu.CompilerParams(
            collective_id=0,
            vmem_limit_bytes=60 * 1024 * 1024,
        ),
    )(x, w_mat)


# device time: 715763 ns/iter; 1.9814x vs baseline; 1.9814x over previous
import jax
import jax.numpy as jnp
from jax import lax
from jax.experimental import pallas as pl
from jax.experimental.pallas import tpu as pltpu

N_DEV = 8
N_RINGS = 4
RING_ORDER = (0, 2, 1, 3)


def _gelu(y):
    c = 0.7978845608028654
    return 0.5 * y * (1.0 + jnp.tanh(c * (y + 0.044715 * y * y * y)))


def kernel(x, w_mat):
    m_global, k_shard = x.shape
    _, n = w_mat.shape
    m_out = m_global // N_DEV
    nq = n // N_RINGS

    def body(x_hbm, w_hbm, out_hbm, x_vmem, w_vmem, send_ref, recv_ref,
             load_sems, w_sem, store_sems, send_sems, recv_sems,
             credit0, credit1, credit2, credit3):
        credit_sems = (credit0, credit1, credit2, credit3)
        my = lax.axis_index("i")
        left = lax.rem(my + N_DEV - 1, N_DEV)
        right = lax.rem(my + 1, N_DEV)
        peers = {0: (right, left), 1: (right, left),
                 2: (left, right), 3: (left, right)}

        barrier = pltpu.get_barrier_semaphore()
        for nbr in (left, right):
            pl.semaphore_signal(
                barrier, inc=1, device_id=(nbr,),
                device_id_type=pl.DeviceIdType.MESH,
            )
        pl.semaphore_wait(barrier, 2)

        w_cp = pltpu.make_async_copy(w_hbm, w_vmem, w_sem)
        w_cp.start()

        def load_chunk(slot, c):
            cp = pltpu.make_async_copy(
                x_hbm.at[pl.ds(c * m_out, m_out), :],
                x_vmem.at[slot], load_sems.at[slot],
            )
            cp.start()
            return cp

        def make_rdma(r):
            dst, _ = peers[r]
            return pltpu.make_async_remote_copy(
                src_ref=send_ref.at[r],
                dst_ref=recv_ref.at[r],
                send_sem=send_sems.at[r],
                recv_sem=recv_sems.at[r],
                device_id=(dst,),
                device_id_type=pl.DeviceIdType.MESH,
            )

        def gemm(r, slot, add_recv, apply_gelu):
            p = jnp.dot(
                x_vmem[slot], w_vmem[:, pl.ds(r * nq, nq)],
                preferred_element_type=jnp.float32,
            )
            if add_recv:
                p = recv_ref[r] + p
            if apply_gelu:
                p = _gelu(p)
            send_ref[r] = p

        cp0 = load_chunk(0, lax.rem(my + N_DEV - 1, N_DEV))
        cp1 = load_chunk(1, lax.rem(my + 1, N_DEV))
        w_cp.wait()
        cp0.wait()
        cp1.wait()
        for r in RING_ORDER:
            gemm(r, 0 if r < 2 else 1, add_recv=False, apply_gelu=False)

        for s in range(N_DEV - 1):
            rdmas = {r: make_rdma(r) for r in RING_ORDER}
            for r in RING_ORDER:
                if s >= 1:
                    pl.semaphore_wait(credit_sems[r], 1)
                rdmas[r].start()
            if s < N_DEV - 2:
                cps = [
                    load_chunk(0, lax.rem(my + 2 * N_DEV - 2 - s, N_DEV)),
                    load_chunk(1, lax.rem(my + 2 + s, N_DEV)),
                ]
            else:
                cps = [load_chunk(0, my)]
            for cp in cps:
                cp.wait()
            for r in RING_ORDER:
                rdmas[r].wait()
                if s < N_DEV - 2:
                    gemm(r, 0 if r < 2 else 1, add_recv=True,
                         apply_gelu=False)
                    _, upstream = peers[r]
                    pl.semaphore_signal(
                        credit_sems[r], inc=1, device_id=(upstream,),
                        device_id_type=pl.DeviceIdType.MESH,
                    )
                else:
                    gemm(r, 0, add_recv=True, apply_gelu=True)
                    pltpu.make_async_copy(
                        send_ref.at[r],
                        out_hbm.at[:, pl.ds(r * nq, nq)],
                        store_sems.at[r],
                    ).start()
        for r in range(N_RINGS):
            pltpu.make_async_copy(
                send_ref.at[r], out_hbm.at[:, pl.ds(r * nq, nq)],
                store_sems.at[r],
            ).wait()

    return pl.pallas_call(
        body,
        out_shape=jax.ShapeDtypeStruct((m_out, n), jnp.float32),
        in_specs=[
            pl.BlockSpec(memory_space=pltpu.MemorySpace.HBM),
            pl.BlockSpec(memory_space=pltpu.MemorySpace.HBM),
        ],
        out_specs=pl.BlockSpec(memory_space=pltpu.MemorySpace.HBM),
        scratch_shapes=[
            pltpu.VMEM((2, m_out, k_shard), jnp.float32),
            pltpu.VMEM((k_shard, n), jnp.float32),
            pltpu.VMEM((N_RINGS, m_out, nq), jnp.float32),
            pltpu.VMEM((N_RINGS, m_out, nq), jnp.float32),
            pltpu.SemaphoreType.DMA((2,)),
            pltpu.SemaphoreType.DMA,
            pltpu.SemaphoreType.DMA((N_RINGS,)),
            pltpu.SemaphoreType.DMA((N_RINGS,)),
            pltpu.SemaphoreType.DMA((N_RINGS,)),
            pltpu.SemaphoreType.REGULAR,
            pltpu.SemaphoreType.REGULAR,
            pltpu.SemaphoreType.REGULAR,
            pltpu.SemaphoreType.REGULAR,
        ],
        compiler_params=pltpu.CompilerParams(
            collective_id=0,
            vmem_limit_bytes=62 * 1024 * 1024,
        ),
    )(x, w_mat)


# device time: 668031 ns/iter; 2.1230x vs baseline; 1.0715x over previous
import jax
import jax.numpy as jnp
from jax import lax
from jax.experimental import pallas as pl
from jax.experimental.pallas import tpu as pltpu

N_DEV = 8
N_RINGS = 4
RING_ORDER = (0, 2, 1, 3)


def _gelu(y):
    c = 0.7978845608028654
    return 0.5 * y * (1.0 + jnp.tanh(c * (y + 0.044715 * y * y * y)))


def kernel(x, w_mat):
    m_global, k_shard = x.shape
    _, n = w_mat.shape
    m_out = m_global // N_DEV
    nq = n // N_RINGS

    def body(x_hbm, w_hbm, out_hbm, x_vmem, w_vmem, send_ref, recv_ref,
             load_sems, w_sem, store_sems, send_sems, recv_sems,
             credit0, credit1, credit2, credit3):
        credit_sems = (credit0, credit1, credit2, credit3)
        my = lax.axis_index("i")
        left = lax.rem(my + N_DEV - 1, N_DEV)
        right = lax.rem(my + 1, N_DEV)
        peers = {0: (right, left), 1: (right, left),
                 2: (left, right), 3: (left, right)}

        barrier = pltpu.get_barrier_semaphore()
        for nbr in (left, right):
            pl.semaphore_signal(
                barrier, inc=1, device_id=(nbr,),
                device_id_type=pl.DeviceIdType.MESH,
            )
        pl.semaphore_wait(barrier, 2)

        w_cp = pltpu.make_async_copy(w_hbm, w_vmem, w_sem)
        w_cp.start()

        def load_chunk(slot, c):
            cp = pltpu.make_async_copy(
                x_hbm.at[pl.ds(c * m_out, m_out), :],
                x_vmem.at[slot], load_sems.at[slot],
            )
            cp.start()
            return cp

        def make_rdma(r):
            dst, _ = peers[r]
            return pltpu.make_async_remote_copy(
                src_ref=send_ref.at[r],
                dst_ref=recv_ref.at[r],
                send_sem=send_sems.at[r],
                recv_sem=recv_sems.at[r],
                device_id=(dst,),
                device_id_type=pl.DeviceIdType.MESH,
            )

        def gemm(r, slot, add_recv, apply_gelu):
            p = jnp.dot(
                x_vmem[slot], w_vmem[:, pl.ds(r * nq, nq)],
                preferred_element_type=jnp.float32,
            )
            if add_recv:
                p = recv_ref[r] + p
            if apply_gelu:
                p = _gelu(p)
            send_ref[r] = p

        cp0 = load_chunk(0, lax.rem(my + N_DEV - 1, N_DEV))
        cp1 = load_chunk(1, lax.rem(my + 1, N_DEV))
        w_cp.wait()
        cp0.wait()
        cp1.wait()
        rdmas = {}
        for r in RING_ORDER:
            gemm(r, 0 if r < 2 else 1, add_recv=False, apply_gelu=False)
            rdmas[r] = make_rdma(r)
            rdmas[r].start()
        cpR = load_chunk(0, lax.rem(my + N_DEV - 2, N_DEV))
        cpL = load_chunk(1, lax.rem(my + 2, N_DEV))

        for s in range(N_DEV - 1):
            last = s == N_DEV - 2
            for r in RING_ORDER:
                rdmas[r].wait()
                if r == 0:
                    cpR.wait()
                elif r == 2:
                    cpL.wait()
                if not last:
                    gemm(r, 0 if r < 2 else 1, add_recv=True,
                         apply_gelu=False)
                    _, upstream = peers[r]
                    pl.semaphore_signal(
                        credit_sems[r], inc=1, device_id=(upstream,),
                        device_id_type=pl.DeviceIdType.MESH,
                    )
                    pl.semaphore_wait(credit_sems[r], 1)
                    rdmas[r] = make_rdma(r)
                    rdmas[r].start()
                    if r == 1:
                        cpR = load_chunk(0, lax.rem(my + 2 * N_DEV - 3 - s,
                                                    N_DEV))
                    elif r == 3:
                        cpL = load_chunk(1, lax.rem(my + 3 + s, N_DEV))
                else:
                    gemm(r, 0 if r < 2 else 1, add_recv=True,
                         apply_gelu=True)
                    pltpu.make_async_copy(
                        send_ref.at[r],
                        out_hbm.at[:, pl.ds(r * nq, nq)],
                        store_sems.at[r],
                    ).start()
        for r in range(N_RINGS):
            pltpu.make_async_copy(
                send_ref.at[r], out_hbm.at[:, pl.ds(r * nq, nq)],
                store_sems.at[r],
            ).wait()

    return pl.pallas_call(
        body,
        out_shape=jax.ShapeDtypeStruct((m_out, n), jnp.float32),
        in_specs=[
            pl.BlockSpec(memory_space=pltpu.MemorySpace.HBM),
            pl.BlockSpec(memory_space=pltpu.MemorySpace.HBM),
        ],
        out_specs=pl.BlockSpec(memory_space=pltpu.MemorySpace.HBM),
        scratch_shapes=[
            pltpu.VMEM((2, m_out, k_shard), jnp.float32),
            pltpu.VMEM((k_shard, n), jnp.float32),
            pltpu.VMEM((N_RINGS, m_out, nq), jnp.float32),
            pltpu.VMEM((N_RINGS, m_out, nq), jnp.float32),
            pltpu.SemaphoreType.DMA((2,)),
            pltpu.SemaphoreType.DMA,
            pltpu.SemaphoreType.DMA((N_RINGS,)),
            pltpu.SemaphoreType.DMA((N_RINGS,)),
            pltpu.SemaphoreType.DMA((N_RINGS,)),
            pltpu.SemaphoreType.REGULAR,
            pltpu.SemaphoreType.REGULAR,
            pltpu.SemaphoreType.REGULAR,
            pltpu.SemaphoreType.REGULAR,
        ],
        compiler_params=pltpu.CompilerParams(
            collective_id=0,
            vmem_limit_bytes=62 * 1024 * 1024,
        ),
    )(x, w_mat)
